# Initial kernel scaffold; baseline (speedup 1.0000x reference)
import jax
import jax.numpy as jnp
from jax.experimental import pallas as pl


def kernel(x, edge_index, edge_weight, W_rel0, b_rel0, W_root0, W_rel1, b_rel1, W_root1, W_rel2, b_rel2, W_root2, W_rel3, b_rel3, W_root3, W_rel4, b_rel4, W_root4):
    raise NotImplementedError("write your pallas kernel here")


# speedup vs baseline: 6.6274x; 6.6274x over previous
import dataclasses
import functools

import jax
import jax.numpy as jnp
from jax import lax
from jax.experimental import pallas as pl
from jax.experimental.pallas import tpu as pltpu
from jax.experimental.pallas import tpu_sc as plsc

_N = 100000
_E = 3200000
_F = 16
_H = 20
_B = 128
_TILES = 32
_CPT = 784
_K = 112
_ROUNDS = _CPT // _K
_R = _TILES * _CPT
_EP = _R * _B
_PH = 4
_NP = 100352
_NH = _NP // _PH
_NPS = _NH // 16

_BR = 2000


def _make_sc_aggregate():
  mesh = plsc.VectorSubcoreMesh(
      core_axis_name="c", subcore_axis_name="s", num_cores=2, num_subcores=16)
  cp = pltpu.CompilerParams()
  if "needs_layout_passes" in pltpu.CompilerParams.__dataclass_fields__:
    cp = dataclasses.replace(cp, needs_layout_passes=False)
  if "use_tc_tiling_on_sc" in pltpu.CompilerParams.__dataclass_fields__:
    cp = dataclasses.replace(cp, use_tc_tiling_on_sc=False)

  @functools.partial(
      pl.kernel,
      out_type=jax.ShapeDtypeStruct((2, _NP, _F), jnp.float32),
      mesh=mesh,
      compiler_params=cp,
      scratch_types=[
          pltpu.VMEM((_K, _B), jnp.int32),
          pltpu.VMEM((_K, _B), jnp.int32),
          pltpu.VMEM((_K, _B), jnp.float32),
          pltpu.VMEM((_B, _F), jnp.float32),
          pltpu.VMEM((_B, _F), jnp.float32),
          pltpu.VMEM((_B, _F), jnp.float32),
          pltpu.VMEM((_B, _F), jnp.float32),
          pltpu.VMEM((_NPS, _F), jnp.float32),
          pltpu.VMEM_SHARED((_NH, _F), jnp.float32),
          pltpu.SemaphoreType.DMA,
          pltpu.SemaphoreType.DMA,
          pltpu.SemaphoreType.DMA,
          pltpu.SemaphoreType.DMA,
          pltpu.SemaphoreType.DMA,
          pltpu.SemaphoreType.DMA,
          pltpu.SemaphoreType.DMA,
          pltpu.SemaphoreType.DMA,
          pltpu.SemaphoreType.DMA,
      ],
  )
  def body(g_hbm, src_hbm, dst_hbm, w_hbm, out_hbm,
           srcs, dsts, ws, r0, r1, r2, r3, zbuf, aggsh,
           stsem, g0s, g1s, g2s, g3s, s0s, s1s, s2s, s3s):
    cid = lax.axis_index("c")
    sid = lax.axis_index("s")
    wid = cid * 16 + sid
    lane_full = [jnp.full((16,), t, jnp.int32) for t in range(16)]
    rowbuf = (r0, r1, r2, r3)
    gsem = (g0s, g1s, g2s, g3s)
    ssem = (s0s, s1s, s2s, s3s)

    @pl.loop(0, _NPS, unroll=8)
    def _(i):
      zbuf[i] = jnp.zeros((_F,), jnp.float32)
    nbase = sid * _NPS
    row0 = wid * _CPT

    def issue_gather(t, c):
      return pltpu.async_copy(
          g_hbm.at[plsc.Indices(srcs.at[c], ignored_value=-1)],
          rowbuf[t], gsem[t])

    def wait_gather(t, c):
      pltpu.make_async_copy(
          g_hbm.at[plsc.Indices(srcs.at[c], ignored_value=-1)],
          rowbuf[t], gsem[t]).wait()

    def issue_scatter(t, c):
      return pltpu.async_copy(
          rowbuf[t],
          aggsh.at[plsc.Indices(dsts.at[c], ignored_value=-1)],
          ssem[t], add=True)

    def wait_scatter(t, c):
      pltpu.make_async_copy(
          rowbuf[t],
          aggsh.at[plsc.Indices(dsts.at[c], ignored_value=-1)],
          ssem[t]).wait()

    @pl.loop(0, _PH)
    def _(ph):
      pltpu.sync_copy(zbuf, aggsh.at[pl.ds(nbase, _NPS)])
      plsc.subcore_barrier()

      @pl.loop(0, _ROUNDS)
      def _(r):
        rbase = row0 + r * _K
        d1 = pltpu.async_copy(src_hbm.at[ph, pl.ds(rbase, _K)], srcs, stsem)
        d2 = pltpu.async_copy(dst_hbm.at[ph, pl.ds(rbase, _K)], dsts, stsem)
        d3 = pltpu.async_copy(w_hbm.at[pl.ds(rbase, _K)], ws, stsem)
        d1.wait()
        d2.wait()
        d3.wait()
        for t in range(4):
          issue_gather(t, t)

        @pl.loop(0, _K // 4)
        def _(q):
          c0 = q * 4
          for t in range(4):
            c = c0 + t
            wait_gather(t, c)
            rt = rowbuf[t]

            cvec = jnp.full((16,), c, jnp.int32)

            @pl.loop(0, _B // 16)
            def _(gq):
              i0 = gq * 16
              i0vec = jnp.full((16,), i0, jnp.int32)
              for t16 in range(16):
                s = plsc.load_gather(ws, [cvec, i0vec + lane_full[t16]])
                rt[i0 + t16, :] = rt[i0 + t16, :] * s

            issue_scatter(t, c)
          for t in range(4):
            wait_scatter(t, c0 + t)

            @pl.when(c0 + 4 + t < _K)
            def _():
              issue_gather(t, c0 + 4 + t)

      plsc.subcore_barrier()
      pltpu.sync_copy(aggsh.at[pl.ds(nbase, _NPS)],
                      out_hbm.at[cid, pl.ds(ph * _NH + nbase, _NPS)])
      plsc.subcore_barrier()

  return body


_sc_aggregate = _make_sc_aggregate()


def _dot_t(a, w):
  return lax.dot_general(a, w, (((1,), (1,)), ((), ())),
                         preferred_element_type=jnp.float32)


def _tc_step(agg, h, p, b, wn, wrn):
  def tc_body(agg_ref, h_ref, p_ref, b_ref, wn_ref, wrn_ref, ho_ref, go_ref):
    a = agg_ref[0] + agg_ref[1]
    z = _dot_t(a, p_ref[...]) + b_ref[...] + _dot_t(h_ref[...], wn_ref[...])
    hn = jnp.maximum(z, 0.0)
    ho_ref[...] = hn
    go_ref[...] = _dot_t(hn, wrn_ref[...])

  return pl.pallas_call(
      tc_body,
      grid=(_N // _BR,),
      in_specs=[
          pl.BlockSpec((2, _BR, _F), lambda i: (0, i, 0)),
          pl.BlockSpec((_BR, _H), lambda i: (i, 0)),
          pl.BlockSpec((_H, _F), lambda i: (0, 0)),
          pl.BlockSpec((1, _H), lambda i: (0, 0)),
          pl.BlockSpec((_H, _H), lambda i: (0, 0)),
          pl.BlockSpec((_F, _H), lambda i: (0, 0)),
      ],
      out_specs=[
          pl.BlockSpec((_BR, _H), lambda i: (i, 0)),
          pl.BlockSpec((_BR, _F), lambda i: (i, 0)),
      ],
      out_shape=[
          jax.ShapeDtypeStruct((_N, _H), jnp.float32),
          jax.ShapeDtypeStruct((_N, _F), jnp.float32),
      ],
  )(agg, h, p, b, wn, wrn)


def _tc_softmax(h):
  def tc_body(h_ref, o_ref):
    z = h_ref[...][:, :2]
    m = jnp.max(z, axis=1, keepdims=True)
    e = jnp.exp(z - m)
    o_ref[...] = e / jnp.sum(e, axis=1, keepdims=True)

  return pl.pallas_call(
      tc_body,
      grid=(_N // _BR,),
      in_specs=[pl.BlockSpec((_BR, _H), lambda i: (i, 0))],
      out_specs=pl.BlockSpec((_BR, 2), lambda i: (i, 0)),
      out_shape=jax.ShapeDtypeStruct((_N, 2), jnp.float32),
  )(h)


def _pad2(w, rows, cols):
  out = jnp.zeros((rows, cols), jnp.float32)
  return out.at[:w.shape[0], :w.shape[1]].set(w)


def kernel(x, edge_index, edge_weight,
           W_rel0, b_rel0, W_root0,
           W_rel1, b_rel1, W_root1,
           W_rel2, b_rel2, W_root2,
           W_rel3, b_rel3, W_root3,
           W_rel4, b_rel4, W_root4):
  pad = _EP - _E
  srcp = jnp.concatenate([edge_index[0], jnp.zeros((pad,), jnp.int32)])
  dstp = jnp.concatenate([edge_index[1], jnp.zeros((pad,), jnp.int32)])
  wp = jnp.concatenate([edge_weight, jnp.zeros((pad,), jnp.float32)])
  neg1 = jnp.int32(-1)
  srcs_ph, dsts_ph = [], []
  for ph in range(_PH):
    live = (dstp >= ph * _NH) & (dstp < (ph + 1) * _NH)
    srcs_ph.append(jnp.where(live, srcp, neg1))
    dsts_ph.append(jnp.where(live, dstp - ph * _NH, neg1))
  src_st = jnp.stack(srcs_ph).reshape(_PH, _R, _B)
  dst_st = jnp.stack(dsts_ph).reshape(_PH, _R, _B)
  w2 = wp.reshape(_R, _B)

  eye = jnp.zeros((_H, _F), jnp.float32).at[:_F, :_F].set(jnp.eye(_F))
  p_all = jnp.stack([_pad2(W_rel0, _H, _F), eye, eye, eye, eye])
  wn_all = jnp.stack([_pad2(w, _H, _H)
                      for w in (W_root0, W_root1, W_root2, W_root3, W_root4)])
  b_all = jnp.stack([_pad2(b.reshape(1, -1), 1, _H)
                     for b in (b_rel0, b_rel1, b_rel2, b_rel3, b_rel4)])
  wrn_all = jnp.stack([_pad2(w, _F, _H)
                       for w in (W_rel1, W_rel2, W_rel3, W_rel4,
                                 jnp.zeros((2, 5), jnp.float32))])

  h0 = _pad2(x, _N, _H)
  g0 = _pad2(x, _N, _F)

  def layer(carry, params):
    h, g = carry
    p, b, wn, wrn = params
    agg = _sc_aggregate(g, src_st, dst_st, w2)
    hn, gn = _tc_step(agg, h, p, b, wn, wrn)
    return (hn, gn), None

  (h5, _), _ = lax.scan(layer, (h0, g0), (p_all, b_all, wn_all, wrn_all))
  return _tc_softmax(h5)
